# baseline (speedup 1.0000x reference)
import functools

import jax
import jax.numpy as jnp
from jax import lax
from jax.experimental import pallas as pl
from jax.experimental.pallas import tpu as pltpu
from jax.experimental.pallas import tpu_sc as plsc

_LANES = 16
_TILES = 16
_NUM_SEGMENTS = 512
_BLK = _NUM_SEGMENTS // _TILES
_STRIDE = _NUM_SEGMENTS + 1
_UNROLL = 8


@functools.lru_cache(maxsize=None)
def _make_seg_sum(nvec_total: int):
    base_vecs = nvec_total // _TILES
    extra = nvec_total % _TILES
    max_vecs = base_vecs + (1 if extra else 0)
    acc16_words = _STRIDE * _LANES
    mesh = plsc.VectorSubcoreMesh(
        core_axis_name="c", subcore_axis_name="s", num_cores=1
    )

    @functools.partial(
        pl.kernel,
        out_type=jax.ShapeDtypeStruct((_NUM_SEGMENTS,), jnp.float32),
        mesh=mesh,
        compiler_params=pltpu.CompilerParams(
            needs_layout_passes=False,
            disable_bounds_checks=True,
            disable_semaphore_checks=True,
        ),
        scratch_types=[
            pltpu.VMEM((max_vecs * _LANES,), jnp.float32),
            pltpu.VMEM((max_vecs * _LANES,), jnp.int32),
            pltpu.VMEM((acc16_words,), jnp.float32),
            pltpu.VMEM((_NUM_SEGMENTS,), jnp.float32),
            pltpu.VMEM((_TILES, _BLK), jnp.float32),
            pltpu.VMEM((_BLK,), jnp.float32),
            pltpu.VMEM_SHARED((_TILES, _NUM_SEGMENTS), jnp.float32),
            pltpu.SemaphoreType.DMA,
            pltpu.SemaphoreType.DMA,
        ],
    )
    def seg_sum(val_hbm, idx_hbm, out_hbm, val_v, idx_v, acc16_v, acc_v,
                colbuf_v, res_v, shared, sem0, sem1):
        wid = lax.axis_index("s")
        base = (wid * base_vecs + jnp.minimum(wid, extra)) * _LANES

        cnt_max = max_vecs * _LANES
        win = jnp.minimum(base, nvec_total * _LANES - cnt_max)
        delta = base - win
        cp0 = pltpu.async_copy(val_hbm.at[pl.ds(win, cnt_max)],
                               val_v.at[pl.ds(0, cnt_max)], sem0)
        cp1 = pltpu.async_copy(idx_hbm.at[pl.ds(win, cnt_max)],
                               idx_v.at[pl.ds(0, cnt_max)], sem1)

        zeros16 = jnp.zeros((_LANES,), jnp.float32)
        iota16 = lax.iota(jnp.int32, _LANES)
        lane_off = iota16 * _STRIDE

        def zbody(j, carry):
            acc16_v[pl.ds(pl.multiple_of(j * _LANES, _LANES), _LANES)] = (
                zeros16)
            return carry

        lax.fori_loop(0, acc16_words // _LANES, zbody, 0, unroll=8)

        cp0.wait()
        cp1.wait()

        def phase1(nvec):
            def go():
                def body(i, carry):
                    off = pl.multiple_of(i * _LANES + delta, _LANES)
                    v = val_v[pl.ds(off, _LANES)]
                    b = idx_v[pl.ds(off, _LANES)]
                    plsc.addupdate_scatter(acc16_v, [b + lane_off], v)
                    return carry

                lax.fori_loop(0, nvec, body, 0, unroll=_UNROLL)

            return go

        if extra:
            pl.when(wid < extra)(phase1(base_vecs + 1))
            pl.when(wid >= extra)(phase1(base_vecs))
        else:
            phase1(base_vecs)()

        def fbody(j, carry):
            seg = pl.multiple_of(j * _LANES, _LANES) + iota16
            s = plsc.load_gather(acc16_v, [seg])
            for l in range(1, _LANES):
                s = s + plsc.load_gather(acc16_v, [seg + l * _STRIDE])
            acc_v[pl.ds(pl.multiple_of(j * _LANES, _LANES), _LANES)] = s
            return carry

        lax.fori_loop(0, _NUM_SEGMENTS // _LANES, fbody, 0, unroll=2)

        pltpu.sync_copy(acc_v, shared.at[wid])
        plsc.subcore_barrier()

        col = pl.multiple_of(wid * _BLK, _BLK)
        cps = [pltpu.async_copy(shared.at[r, pl.ds(col, _BLK)],
                                colbuf_v.at[r], sem0)
               for r in range(_TILES)]
        for cp in cps:
            cp.wait()

        a0 = zeros16
        a1 = zeros16
        for r in range(_TILES):
            a0 = a0 + colbuf_v[r, pl.ds(0, _LANES)]
            a1 = a1 + colbuf_v[r, pl.ds(_LANES, _LANES)]
        res_v[pl.ds(0, _LANES)] = a0
        res_v[pl.ds(_LANES, _LANES)] = a1
        pltpu.sync_copy(res_v, out_hbm.at[pl.ds(col, _BLK)])

    return seg_sum


def kernel(atomic_energy, batch):
    n = atomic_energy.shape[0]
    src = jnp.squeeze(atomic_energy, axis=1)
    rem = n % _LANES
    if rem:
        pad = _LANES - rem
        src = jnp.pad(src, (0, pad))
        batch = jnp.pad(batch, (0, pad), constant_values=_NUM_SEGMENTS - 1)
        n += pad
    return _make_seg_sum(n // _LANES)(src, batch)


# speedup vs baseline: 1.0045x; 1.0045x over previous
import functools

import jax
import jax.numpy as jnp
from jax import lax
from jax.experimental import pallas as pl
from jax.experimental.pallas import tpu as pltpu
from jax.experimental.pallas import tpu_sc as plsc

_LANES = 16
_TILES = 16
_NUM_SEGMENTS = 512
_BLK = _NUM_SEGMENTS // _TILES
_STRIDE = _NUM_SEGMENTS + 1
_UNROLL = 8


@functools.lru_cache(maxsize=None)
def _make_seg_sum(nvec_total: int):
    base_vecs = nvec_total // _TILES
    extra = nvec_total % _TILES
    max_vecs = base_vecs + (1 if extra else 0)
    acc16_words = _STRIDE * _LANES
    mesh = plsc.VectorSubcoreMesh(
        core_axis_name="c", subcore_axis_name="s", num_cores=1
    )

    @functools.partial(
        pl.kernel,
        out_type=jax.ShapeDtypeStruct((_NUM_SEGMENTS,), jnp.float32),
        mesh=mesh,
        compiler_params=pltpu.CompilerParams(
            needs_layout_passes=False,
            disable_bounds_checks=True,
            disable_semaphore_checks=True,
            use_tc_tiling_on_sc=False,
        ),
        scratch_types=[
            pltpu.VMEM((max_vecs * _LANES,), jnp.float32),
            pltpu.VMEM((max_vecs * _LANES,), jnp.int32),
            pltpu.VMEM((acc16_words,), jnp.float32),
            pltpu.VMEM((_NUM_SEGMENTS,), jnp.float32),
            pltpu.VMEM((_TILES, _BLK), jnp.float32),
            pltpu.VMEM((_BLK,), jnp.float32),
            pltpu.VMEM_SHARED((_TILES, _NUM_SEGMENTS), jnp.float32),
            pltpu.SemaphoreType.DMA,
            pltpu.SemaphoreType.DMA,
        ],
    )
    def seg_sum(val_hbm, idx_hbm, out_hbm, val_v, idx_v, acc16_v, acc_v,
                colbuf_v, res_v, shared, sem0, sem1):
        wid = lax.axis_index("s")
        base = (wid * base_vecs + jnp.minimum(wid, extra)) * _LANES

        cnt_max = max_vecs * _LANES
        win = jnp.minimum(base, nvec_total * _LANES - cnt_max)
        delta = base - win
        cp0 = pltpu.async_copy(val_hbm.at[pl.ds(win, cnt_max)],
                               val_v.at[pl.ds(0, cnt_max)], sem0)
        cp1 = pltpu.async_copy(idx_hbm.at[pl.ds(win, cnt_max)],
                               idx_v.at[pl.ds(0, cnt_max)], sem1)

        zeros16 = jnp.zeros((_LANES,), jnp.float32)
        iota16 = lax.iota(jnp.int32, _LANES)
        lane_off = iota16 * _STRIDE

        def zbody(j, carry):
            acc16_v[pl.ds(pl.multiple_of(j * _LANES, _LANES), _LANES)] = (
                zeros16)
            return carry

        lax.fori_loop(0, acc16_words // _LANES, zbody, 0, unroll=8)

        cp0.wait()
        cp1.wait()

        def phase1(nvec):
            def go():
                def body(i, carry):
                    off = pl.multiple_of(i * _LANES + delta, _LANES)
                    v = val_v[pl.ds(off, _LANES)]
                    b = idx_v[pl.ds(off, _LANES)]
                    plsc.addupdate_scatter(acc16_v, [b + lane_off], v)
                    return carry

                lax.fori_loop(0, nvec, body, 0, unroll=_UNROLL)

            return go

        if extra:
            pl.when(wid < extra)(phase1(base_vecs + 1))
            pl.when(wid >= extra)(phase1(base_vecs))
        else:
            phase1(base_vecs)()

        def fbody(j, carry):
            seg = pl.multiple_of(j * _LANES, _LANES) + iota16
            s = plsc.load_gather(acc16_v, [seg])
            for l in range(1, _LANES):
                s = s + plsc.load_gather(acc16_v, [seg + l * _STRIDE])
            acc_v[pl.ds(pl.multiple_of(j * _LANES, _LANES), _LANES)] = s
            return carry

        lax.fori_loop(0, _NUM_SEGMENTS // _LANES, fbody, 0, unroll=2)

        pltpu.sync_copy(acc_v, shared.at[wid])
        plsc.subcore_barrier()

        col = pl.multiple_of(wid * _BLK, _BLK)
        cps = [pltpu.async_copy(shared.at[r, pl.ds(col, _BLK)],
                                colbuf_v.at[r], sem0)
               for r in range(_TILES)]
        for cp in cps:
            cp.wait()

        a0 = zeros16
        a1 = zeros16
        for r in range(_TILES):
            a0 = a0 + colbuf_v[r, pl.ds(0, _LANES)]
            a1 = a1 + colbuf_v[r, pl.ds(_LANES, _LANES)]
        res_v[pl.ds(0, _LANES)] = a0
        res_v[pl.ds(_LANES, _LANES)] = a1
        pltpu.sync_copy(res_v, out_hbm.at[pl.ds(col, _BLK)])

    return seg_sum


def kernel(atomic_energy, batch):
    n = atomic_energy.shape[0]
    src = jnp.squeeze(atomic_energy, axis=1)
    rem = n % _LANES
    if rem:
        pad = _LANES - rem
        src = jnp.pad(src, (0, pad))
        batch = jnp.pad(batch, (0, pad), constant_values=_NUM_SEGMENTS - 1)
        n += pad
    return _make_seg_sum(n // _LANES)(src, batch)


# speedup vs baseline: 1.0173x; 1.0128x over previous
import functools

import jax
import jax.numpy as jnp
from jax import lax
from jax.experimental import pallas as pl
from jax.experimental.pallas import tpu as pltpu
from jax.experimental.pallas import tpu_sc as plsc

_LANES = 16
_TILES = 16
_NUM_SEGMENTS = 512
_BLK = _NUM_SEGMENTS // _TILES
_SUBACCS = 8
_STRIDE = _NUM_SEGMENTS + 3
_UNROLL = 8


@functools.lru_cache(maxsize=None)
def _make_seg_sum(nvec_total: int):
    base_vecs = nvec_total // _TILES
    extra = nvec_total % _TILES
    max_vecs = base_vecs + (1 if extra else 0)
    acc8_words = -(-_STRIDE * _SUBACCS // _LANES) * _LANES
    mesh = plsc.VectorSubcoreMesh(
        core_axis_name="c", subcore_axis_name="s", num_cores=1
    )

    @functools.partial(
        pl.kernel,
        out_type=jax.ShapeDtypeStruct((_NUM_SEGMENTS,), jnp.float32),
        mesh=mesh,
        compiler_params=pltpu.CompilerParams(
            needs_layout_passes=False,
            disable_bounds_checks=True,
            disable_semaphore_checks=True,
            use_tc_tiling_on_sc=False,
        ),
        scratch_types=[
            pltpu.VMEM((max_vecs * _LANES,), jnp.float32),
            pltpu.VMEM((max_vecs * _LANES,), jnp.int32),
            pltpu.VMEM((acc8_words,), jnp.float32),
            pltpu.VMEM((_NUM_SEGMENTS,), jnp.float32),
            pltpu.VMEM((_TILES, _BLK), jnp.float32),
            pltpu.VMEM((_BLK,), jnp.float32),
            pltpu.VMEM_SHARED((_TILES, _NUM_SEGMENTS), jnp.float32),
            pltpu.SemaphoreType.DMA,
            pltpu.SemaphoreType.DMA,
        ],
    )
    def seg_sum(val_hbm, idx_hbm, out_hbm, val_v, idx_v, acc8_v, acc_v,
                colbuf_v, res_v, shared, sem0, sem1):
        wid = lax.axis_index("s")
        base = (wid * base_vecs + jnp.minimum(wid, extra)) * _LANES

        cnt_max = max_vecs * _LANES
        half = (max_vecs // 2) * _LANES
        win = jnp.minimum(base, nvec_total * _LANES - cnt_max)
        delta = base - win
        cp0 = pltpu.async_copy(val_hbm.at[pl.ds(win, half)],
                               val_v.at[pl.ds(0, half)], sem0)
        cp1 = pltpu.async_copy(idx_hbm.at[pl.ds(win, half)],
                               idx_v.at[pl.ds(0, half)], sem0)
        rest = cnt_max - half
        cp2 = pltpu.async_copy(val_hbm.at[pl.ds(win + half, rest)],
                               val_v.at[pl.ds(half, rest)], sem1)
        cp3 = pltpu.async_copy(idx_hbm.at[pl.ds(win + half, rest)],
                               idx_v.at[pl.ds(half, rest)], sem1)

        zeros16 = jnp.zeros((_LANES,), jnp.float32)
        iota16 = lax.iota(jnp.int32, _LANES)
        lane_off = (iota16 & (_SUBACCS - 1)) * _STRIDE

        def zbody(j, carry):
            acc8_v[pl.ds(pl.multiple_of(j * _LANES, _LANES), _LANES)] = (
                zeros16)
            return carry

        lax.fori_loop(0, acc8_words // _LANES, zbody, 0, unroll=8)

        cp0.wait()
        cp1.wait()
        cp2.wait()
        cp3.wait()

        def body(i, carry):
            off = pl.multiple_of(i * _LANES + delta, _LANES)
            v = val_v[pl.ds(off, _LANES)]
            b = idx_v[pl.ds(off, _LANES)]
            plsc.addupdate_scatter(acc8_v, [b + lane_off], v)
            return carry

        lax.fori_loop(0, base_vecs, body, 0, unroll=_UNROLL)
        if extra:

            def _extra_iter():
                body(base_vecs, 0)

            pl.when(wid < extra)(_extra_iter)

        def fbody(j, carry):
            seg = pl.multiple_of(j * _LANES, _LANES) + iota16
            s = plsc.load_gather(acc8_v, [seg])
            for l in range(1, _SUBACCS):
                s = s + plsc.load_gather(acc8_v, [seg + l * _STRIDE])
            acc_v[pl.ds(pl.multiple_of(j * _LANES, _LANES), _LANES)] = s
            return carry

        lax.fori_loop(0, _NUM_SEGMENTS // _LANES, fbody, 0, unroll=2)

        pltpu.sync_copy(acc_v, shared.at[wid])
        plsc.subcore_barrier()

        col = pl.multiple_of(wid * _BLK, _BLK)
        cps = [pltpu.async_copy(shared.at[r, pl.ds(col, _BLK)],
                                colbuf_v.at[r], sem0)
               for r in range(_TILES)]
        for cp in cps:
            cp.wait()

        a0 = zeros16
        a1 = zeros16
        for r in range(_TILES):
            a0 = a0 + colbuf_v[r, pl.ds(0, _LANES)]
            a1 = a1 + colbuf_v[r, pl.ds(_LANES, _LANES)]
        res_v[pl.ds(0, _LANES)] = a0
        res_v[pl.ds(_LANES, _LANES)] = a1
        pltpu.sync_copy(res_v, out_hbm.at[pl.ds(col, _BLK)])

    return seg_sum


def kernel(atomic_energy, batch):
    n = atomic_energy.shape[0]
    src = jnp.squeeze(atomic_energy, axis=1)
    rem = n % _LANES
    if rem:
        pad = _LANES - rem
        src = jnp.pad(src, (0, pad))
        batch = jnp.pad(batch, (0, pad), constant_values=_NUM_SEGMENTS - 1)
        n += pad
    return _make_seg_sum(n // _LANES)(src, batch)


# speedup vs baseline: 1.1049x; 1.0860x over previous
import functools

import jax
import jax.numpy as jnp
from jax import lax
from jax.experimental import pallas as pl
from jax.experimental.pallas import tpu as pltpu
from jax.experimental.pallas import tpu_sc as plsc

_LANES = 16
_TILES = 16
_NUM_SEGMENTS = 512
_BLK = _NUM_SEGMENTS // _TILES
_SUBACCS = 8
_STRIDE = _NUM_SEGMENTS + 3
_UNROLL = 8


@functools.lru_cache(maxsize=None)
def _make_seg_sum(nvec_total: int):
    base_vecs = nvec_total // _TILES
    extra = nvec_total % _TILES
    max_vecs = base_vecs + (1 if extra else 0)
    acc8_words = -(-_STRIDE * _SUBACCS // _LANES) * _LANES
    mesh = plsc.VectorSubcoreMesh(
        core_axis_name="c", subcore_axis_name="s", num_cores=1
    )

    @functools.partial(
        pl.kernel,
        out_type=jax.ShapeDtypeStruct((_NUM_SEGMENTS,), jnp.float32),
        mesh=mesh,
        compiler_params=pltpu.CompilerParams(
            needs_layout_passes=False,
            disable_bounds_checks=True,
            disable_semaphore_checks=True,
            use_tc_tiling_on_sc=False,
        ),
        scratch_types=[
            pltpu.VMEM((max_vecs * _LANES,), jnp.float32),
            pltpu.VMEM((max_vecs * _LANES,), jnp.int32),
            pltpu.VMEM((acc8_words,), jnp.float32),
            pltpu.VMEM((_NUM_SEGMENTS,), jnp.float32),
            pltpu.VMEM((_TILES, _BLK), jnp.float32),
            pltpu.VMEM((_BLK,), jnp.float32),
            pltpu.VMEM_SHARED((_TILES, _NUM_SEGMENTS), jnp.float32),
            pltpu.SemaphoreType.DMA,
            pltpu.SemaphoreType.DMA,
        ],
    )
    def seg_sum(val_hbm, idx_hbm, out_hbm, val_v, idx_v, acc8_v, acc_v,
                colbuf_v, res_v, shared, sem0, sem1):
        wid = lax.axis_index("s")
        base = (wid * base_vecs + jnp.minimum(wid, extra)) * _LANES

        cnt_max = max_vecs * _LANES
        half = (max_vecs // 2) * _LANES
        win = jnp.minimum(base, nvec_total * _LANES - cnt_max)
        delta = base - win
        cp0 = pltpu.async_copy(val_hbm.at[pl.ds(win, half)],
                               val_v.at[pl.ds(0, half)], sem0)
        cp1 = pltpu.async_copy(idx_hbm.at[pl.ds(win, half)],
                               idx_v.at[pl.ds(0, half)], sem0)
        rest = cnt_max - half
        cp2 = pltpu.async_copy(val_hbm.at[pl.ds(win + half, rest)],
                               val_v.at[pl.ds(half, rest)], sem1)
        cp3 = pltpu.async_copy(idx_hbm.at[pl.ds(win + half, rest)],
                               idx_v.at[pl.ds(half, rest)], sem1)

        zeros16 = jnp.zeros((_LANES,), jnp.float32)
        iota16 = lax.iota(jnp.int32, _LANES)
        lane_off = (iota16 & (_SUBACCS - 1)) * _STRIDE

        @plsc.parallel_loop(0, acc8_words, step=_LANES, unroll=8)
        def _zero(j):
            acc8_v[pl.ds(pl.multiple_of(j, _LANES), _LANES)] = zeros16

        cp0.wait()
        cp1.wait()
        cp2.wait()
        cp3.wait()

        def scat(off):
            off = pl.multiple_of(off, _LANES)
            v = val_v[pl.ds(off, _LANES)]
            b = idx_v[pl.ds(off, _LANES)]
            plsc.addupdate_scatter(acc8_v, [b + lane_off], v)

        @plsc.parallel_loop(0, base_vecs * _LANES, step=_LANES,
                            unroll=_UNROLL)
        def _scatter(i):
            scat(i + delta)

        if extra:
            pl.when(wid < extra)(
                functools.partial(scat, base_vecs * _LANES + delta))

        @plsc.parallel_loop(0, _NUM_SEGMENTS, step=_LANES, unroll=2)
        def _fold(j):
            seg = pl.multiple_of(j, _LANES) + iota16
            s = plsc.load_gather(acc8_v, [seg])
            for l in range(1, _SUBACCS):
                s = s + plsc.load_gather(acc8_v, [seg + l * _STRIDE])
            acc_v[pl.ds(pl.multiple_of(j, _LANES), _LANES)] = s

        pltpu.sync_copy(acc_v, shared.at[wid])
        plsc.subcore_barrier()

        col = pl.multiple_of(wid * _BLK, _BLK)
        cps = [pltpu.async_copy(shared.at[r, pl.ds(col, _BLK)],
                                colbuf_v.at[r], sem0)
               for r in range(_TILES)]
        for cp in cps:
            cp.wait()

        a0 = zeros16
        a1 = zeros16
        for r in range(_TILES):
            a0 = a0 + colbuf_v[r, pl.ds(0, _LANES)]
            a1 = a1 + colbuf_v[r, pl.ds(_LANES, _LANES)]
        res_v[pl.ds(0, _LANES)] = a0
        res_v[pl.ds(_LANES, _LANES)] = a1
        pltpu.sync_copy(res_v, out_hbm.at[pl.ds(col, _BLK)])

    return seg_sum


def kernel(atomic_energy, batch):
    n = atomic_energy.shape[0]
    src = jnp.squeeze(atomic_energy, axis=1)
    rem = n % _LANES
    if rem:
        pad = _LANES - rem
        src = jnp.pad(src, (0, pad))
        batch = jnp.pad(batch, (0, pad), constant_values=_NUM_SEGMENTS - 1)
        n += pad
    return _make_seg_sum(n // _LANES)(src, batch)
